# Initial kernel scaffold; baseline (speedup 1.0000x reference)
import functools

import jax
import jax.numpy as jnp
from jax import lax
from jax.experimental import pallas as pl
from jax.experimental.pallas import tpu as pltpu
from jax.experimental.pallas import tpu_sc as plsc

N_EDGES = 320000
D = 128
N_OUT = 10000
NC = 2
NS = 16
EDGES_PER_TILE = N_EDGES // (NC * NS)
C = 80
N_CHUNKS = EDGES_PER_TILE // C
ROWS_PER_TILE = N_OUT // NS
ZR = 125


def _sc_body(src_hbm, idx_hbm, out_hbm, idx_v, row_v, zero_v, acc):
    cid = lax.axis_index("c")
    sid = lax.axis_index("s")
    base = (cid * NS + sid) * EDGES_PER_TILE

    z = jnp.zeros((16,), jnp.float32)

    def zero_row(r, carry):
        for j in range(D // 16):
            zero_v[r, pl.ds(16 * j, 16)] = z
        return carry

    lax.fori_loop(0, ZR, zero_row, 0)
    row0 = sid * ROWS_PER_TILE
    for k in range(ROWS_PER_TILE // ZR):
        pltpu.sync_copy(zero_v, acc.at[pl.ds(row0 + k * ZR, ZR)])
    plsc.subcore_barrier()

    def step(it, carry):
        e0 = base + it * C
        pltpu.sync_copy(idx_hbm.at[pl.ds(e0, C)], idx_v)
        pltpu.sync_copy(src_hbm.at[pl.ds(e0, C)], row_v)
        pltpu.sync_copy(row_v, acc.at[idx_v], add=True)
        return carry

    lax.fori_loop(0, N_CHUNKS, step, 0)
    plsc.subcore_barrier()

    pltpu.sync_copy(acc.at[pl.ds(row0, ROWS_PER_TILE)],
                    out_hbm.at[cid].at[pl.ds(row0, ROWS_PER_TILE)])


@jax.jit
def _sc_scatter(src, idx):
    mesh = plsc.VectorSubcoreMesh(core_axis_name="c", subcore_axis_name="s")
    return pl.kernel(
        _sc_body,
        out_type=jax.ShapeDtypeStruct((NC, N_OUT, D), jnp.float32),
        mesh=mesh,
        scratch_types=[
            pltpu.VMEM((C,), jnp.int32),
            pltpu.VMEM((C, D), jnp.float32),
            pltpu.VMEM((ZR, D), jnp.float32),
            pltpu.VMEM_SHARED((N_OUT, D), jnp.float32),
        ],
    )(src, idx)


def _combine_body(p_ref, o_ref):
    o_ref[...] = p_ref[0] + p_ref[1]


@jax.jit
def _combine(partials):
    blk = 1000
    return pl.pallas_call(
        _combine_body,
        grid=(N_OUT // blk,),
        in_specs=[pl.BlockSpec((NC, blk, D), lambda i: (0, i, 0))],
        out_specs=pl.BlockSpec((blk, D), lambda i: (i, 0)),
        out_shape=jax.ShapeDtypeStruct((N_OUT, D), jnp.float32),
    )(partials)


def kernel(src, index, dim_size):
    idx = index.astype(jnp.int32)
    partials = _sc_scatter(src, idx)
    return _combine(partials)


# speedup vs baseline: 3.7900x; 3.7900x over previous
import functools

import jax
import jax.numpy as jnp
from jax import lax
from jax.experimental import pallas as pl
from jax.experimental.pallas import tpu as pltpu
from jax.experimental.pallas import tpu_sc as plsc

N_EDGES = 320000
D = 128
N_OUT = 10000
NC = 2
NS = 16
EDGES_PER_TILE = N_EDGES // (NC * NS)
C = 80
N_CHUNKS = EDGES_PER_TILE // C
ZR = 80
N_UNITS = N_OUT // ZR


def _sc_body(src_hbm, idx_hbm, out_hbm, idx_v, row_v, zero_v, acc):
    cid = lax.axis_index("c")
    sid = lax.axis_index("s")
    base = (cid * NS + sid) * EDGES_PER_TILE

    z = jnp.zeros((16,), jnp.float32)

    def zero_row(r, carry):
        for j in range(D // 16):
            zero_v[r, pl.ds(16 * j, 16)] = z
        return carry

    lax.fori_loop(0, ZR, zero_row, 0)
    for k in range(pl.cdiv(N_UNITS, NS)):
        u = sid + k * NS

        @pl.when(u < N_UNITS)
        def _():
            r = pl.multiple_of(u * ZR, ZR)
            pltpu.sync_copy(zero_v, acc.at[pl.ds(r, ZR)])

    plsc.subcore_barrier()

    def step(it, carry):
        e0 = base + it * C
        pltpu.sync_copy(idx_hbm.at[pl.ds(e0, C)], idx_v)
        pltpu.sync_copy(src_hbm.at[pl.ds(e0, C)], row_v)
        pltpu.sync_copy(row_v, acc.at[idx_v], add=True)
        return carry

    lax.fori_loop(0, N_CHUNKS, step, 0)
    plsc.subcore_barrier()

    for k in range(pl.cdiv(N_UNITS, NS)):
        u = sid + k * NS

        @pl.when(u < N_UNITS)
        def _():
            r = pl.multiple_of(u * ZR, ZR)
            pltpu.sync_copy(acc.at[pl.ds(r, ZR)],
                            out_hbm.at[cid].at[pl.ds(r, ZR)])


@jax.jit
def _sc_scatter(src, idx):
    mesh = plsc.VectorSubcoreMesh(core_axis_name="c", subcore_axis_name="s")
    return pl.kernel(
        _sc_body,
        out_type=jax.ShapeDtypeStruct((NC, N_OUT, D), jnp.float32),
        mesh=mesh,
        scratch_types=[
            pltpu.VMEM((C,), jnp.int32),
            pltpu.VMEM((C, D), jnp.float32),
            pltpu.VMEM((ZR, D), jnp.float32),
            pltpu.VMEM_SHARED((N_OUT, D), jnp.float32),
        ],
    )(src, idx)


def _combine_body(p_ref, o_ref):
    o_ref[...] = p_ref[0] + p_ref[1]


@jax.jit
def _combine(partials):
    blk = 1000
    return pl.pallas_call(
        _combine_body,
        grid=(N_OUT // blk,),
        in_specs=[pl.BlockSpec((NC, blk, D), lambda i: (0, i, 0))],
        out_specs=pl.BlockSpec((blk, D), lambda i: (i, 0)),
        out_shape=jax.ShapeDtypeStruct((N_OUT, D), jnp.float32),
    )(partials)


def kernel(src, index, dim_size):
    idx = index.astype(jnp.int32)
    partials = _sc_scatter(src, idx)
    return _combine(partials)


# speedup vs baseline: 5.9601x; 1.5726x over previous
import functools

import jax
import jax.numpy as jnp
from jax import lax
from jax.experimental import pallas as pl
from jax.experimental.pallas import tpu as pltpu
from jax.experimental.pallas import tpu_sc as plsc

N_EDGES = 320000
D = 128
N_OUT = 10000
NC = 2
NS = 16
EDGES_PER_TILE = N_EDGES // (NC * NS)
C = 80
N_CHUNKS = EDGES_PER_TILE // C
ZR = 80
N_UNITS = N_OUT // ZR


def _sc_body(src_hbm, idx_hbm, out_hbm, idx_v, row_v0, row_v1, zero_v, acc,
             sem0, sem1):
    cid = lax.axis_index("c")
    sid = lax.axis_index("s")
    wid = cid * NS + sid
    base = wid * EDGES_PER_TILE

    z = jnp.zeros((16,), jnp.float32)

    def zero_row(r, carry):
        for j in range(D // 16):
            zero_v[r, pl.ds(16 * j, 16)] = z
        return carry

    lax.fori_loop(0, ZR, zero_row, 0)
    for k in range(pl.cdiv(N_UNITS, NS)):
        u = sid + k * NS

        @pl.when(u < N_UNITS)
        def _():
            r = pl.multiple_of(u * ZR, ZR)
            pltpu.sync_copy(zero_v, acc.at[pl.ds(r, ZR)])

    pltpu.sync_copy(idx_hbm.at[wid], idx_v)
    plsc.subcore_barrier()

    def load(it, buf, sem):
        pltpu.async_copy(src_hbm.at[pl.ds(base + it * C, C)], buf, sem)

    def drain(buf, sem):
        pltpu.make_async_copy(src_hbm.at[pl.ds(0, C)], buf, sem).wait()

    def scatter(it, buf):
        pltpu.sync_copy(buf, acc.at[idx_v.at[it]], add=True)

    load(0, row_v0, sem0)

    def step(it, carry):
        nxt = it + 1

        def one(cur_buf, cur_sem, other_buf, other_sem):
            drain(cur_buf, cur_sem)

            @pl.when(nxt < N_CHUNKS)
            def _():
                load(nxt, other_buf, other_sem)

            scatter(it, cur_buf)

        @pl.when(it % 2 == 0)
        def _():
            one(row_v0, sem0, row_v1, sem1)

        @pl.when(it % 2 == 1)
        def _():
            one(row_v1, sem1, row_v0, sem0)

        return carry

    lax.fori_loop(0, N_CHUNKS, step, 0)
    plsc.subcore_barrier()

    for k in range(pl.cdiv(N_UNITS, NS)):
        u = sid + k * NS

        @pl.when(u < N_UNITS)
        def _():
            r = pl.multiple_of(u * ZR, ZR)
            pltpu.sync_copy(acc.at[pl.ds(r, ZR)],
                            out_hbm.at[cid].at[pl.ds(r, ZR)])


@jax.jit
def _sc_scatter(src, idx):
    mesh = plsc.VectorSubcoreMesh(core_axis_name="c", subcore_axis_name="s")
    return pl.kernel(
        _sc_body,
        out_type=jax.ShapeDtypeStruct((NC, N_OUT, D), jnp.float32),
        mesh=mesh,
        scratch_types=[
            pltpu.VMEM((N_CHUNKS, C), jnp.int32),
            pltpu.VMEM((C, D), jnp.float32),
            pltpu.VMEM((C, D), jnp.float32),
            pltpu.VMEM((ZR, D), jnp.float32),
            pltpu.VMEM_SHARED((N_OUT, D), jnp.float32),
            pltpu.SemaphoreType.DMA,
            pltpu.SemaphoreType.DMA,
        ],
    )(src, idx.reshape(NC * NS, N_CHUNKS, C))


def _combine_body(p_ref, o_ref):
    o_ref[...] = p_ref[0] + p_ref[1]


@jax.jit
def _combine(partials):
    blk = 1000
    return pl.pallas_call(
        _combine_body,
        grid=(N_OUT // blk,),
        in_specs=[pl.BlockSpec((NC, blk, D), lambda i: (0, i, 0))],
        out_specs=pl.BlockSpec((blk, D), lambda i: (i, 0)),
        out_shape=jax.ShapeDtypeStruct((N_OUT, D), jnp.float32),
    )(partials)


def kernel(src, index, dim_size):
    idx = index.astype(jnp.int32)
    partials = _sc_scatter(src, idx)
    return _combine(partials)


# speedup vs baseline: 8.1592x; 1.3690x over previous
import functools

import jax
import jax.numpy as jnp
from jax import lax
from jax.experimental import pallas as pl
from jax.experimental.pallas import tpu as pltpu
from jax.experimental.pallas import tpu_sc as plsc

N_EDGES = 320000
D = 128
N_OUT = 10000
NC = 2
NS = 16
EDGES_PER_TILE = N_EDGES // (NC * NS)
C = 80
N_CHUNKS = EDGES_PER_TILE // C
ZR = 80
N_UNITS = N_OUT // ZR


def _sc_body(src_hbm, idx_hbm, out_hbm, idx_v, row_v0, row_v1, row_v2,
             acc, sem0, sem1, sem2):
    cid = lax.axis_index("c")
    sid = lax.axis_index("s")
    wid = cid * NS + sid
    base = wid * EDGES_PER_TILE

    zero_v = row_v2
    z = jnp.zeros((16,), jnp.float32)

    def zero_row(r, carry):
        for j in range(D // 16):
            zero_v[r, pl.ds(16 * j, 16)] = z
        return carry

    lax.fori_loop(0, ZR, zero_row, 0)
    for k in range(pl.cdiv(N_UNITS, NS)):
        u = sid + k * NS

        @pl.when(u < N_UNITS)
        def _():
            r = pl.multiple_of(u * ZR, ZR)
            pltpu.sync_copy(zero_v, acc.at[pl.ds(r, ZR)])

    pltpu.sync_copy(idx_hbm.at[wid], idx_v)
    plsc.subcore_barrier()

    def load(it, buf, sem):
        pltpu.async_copy(src_hbm.at[pl.ds(base + it * C, C)], buf, sem)

    def drain(buf, sem):
        pltpu.make_async_copy(src_hbm.at[pl.ds(0, C)], buf, sem).wait()

    def scatter(it, buf):
        pltpu.sync_copy(buf, acc.at[idx_v.at[it]], add=True)

    load(0, row_v0, sem0)
    load(1, row_v1, sem1)

    def step(it, carry):
        def one(cur_buf, cur_sem, prev_buf, prev_sem):
            drain(cur_buf, cur_sem)

            @pl.when(it + 2 < N_CHUNKS)
            def _():
                load(it + 2, prev_buf, prev_sem)

            scatter(it, cur_buf)

        @pl.when(it % 3 == 0)
        def _():
            one(row_v0, sem0, row_v2, sem2)

        @pl.when(it % 3 == 1)
        def _():
            one(row_v1, sem1, row_v0, sem0)

        @pl.when(it % 3 == 2)
        def _():
            one(row_v2, sem2, row_v1, sem1)

        return carry

    lax.fori_loop(0, N_CHUNKS, step, 0)
    plsc.subcore_barrier()

    for k in range(pl.cdiv(N_UNITS, NS)):
        u = sid + k * NS

        @pl.when(u < N_UNITS)
        def _():
            r = pl.multiple_of(u * ZR, ZR)
            pltpu.sync_copy(acc.at[pl.ds(r, ZR)],
                            out_hbm.at[cid].at[pl.ds(r, ZR)])


@jax.jit
def _sc_scatter(src, idx):
    mesh = plsc.VectorSubcoreMesh(core_axis_name="c", subcore_axis_name="s")
    return pl.kernel(
        _sc_body,
        out_type=jax.ShapeDtypeStruct((NC, N_OUT, D), jnp.float32),
        mesh=mesh,
        scratch_types=[
            pltpu.VMEM((N_CHUNKS, C), jnp.int32),
            pltpu.VMEM((C, D), jnp.float32),
            pltpu.VMEM((C, D), jnp.float32),
            pltpu.VMEM((C, D), jnp.float32),
            pltpu.VMEM_SHARED((N_OUT, D), jnp.float32),
            pltpu.SemaphoreType.DMA,
            pltpu.SemaphoreType.DMA,
            pltpu.SemaphoreType.DMA,
        ],
    )(src, idx.reshape(NC * NS, N_CHUNKS, C))


def _combine_body(p_ref, o_ref):
    o_ref[...] = p_ref[0] + p_ref[1]


@jax.jit
def _combine(partials):
    blk = 1000
    return pl.pallas_call(
        _combine_body,
        grid=(N_OUT // blk,),
        in_specs=[pl.BlockSpec((NC, blk, D), lambda i: (0, i, 0))],
        out_specs=pl.BlockSpec((blk, D), lambda i: (i, 0)),
        out_shape=jax.ShapeDtypeStruct((N_OUT, D), jnp.float32),
    )(partials)


def kernel(src, index, dim_size):
    idx = index.astype(jnp.int32)
    partials = _sc_scatter(src, idx)
    return _combine(partials)


# speedup vs baseline: 8.4464x; 1.0352x over previous
import functools

import jax
import jax.numpy as jnp
from jax import lax
from jax.experimental import pallas as pl
from jax.experimental.pallas import tpu as pltpu
from jax.experimental.pallas import tpu_sc as plsc

N_EDGES = 320000
D = 128
N_OUT = 10000
NC = 2
NS = 16
EDGES_PER_TILE = N_EDGES // (NC * NS)
C = 80
N_CHUNKS = EDGES_PER_TILE // C
ZR = 80
N_UNITS = N_OUT // ZR


def _sc_body(src_hbm, idx_hbm, out_hbm, idx_v, row_v0, row_v1, row_v2,
             acc, sem0, sem1, sem2):
    cid = lax.axis_index("c")
    sid = lax.axis_index("s")
    wid = cid * NS + sid
    base = wid * EDGES_PER_TILE

    pltpu.async_copy(src_hbm.at[pl.ds(base, C)], row_v0, sem0)
    pltpu.async_copy(src_hbm.at[pl.ds(base + C, C)], row_v1, sem1)

    zero_v = row_v2
    z = jnp.zeros((16,), jnp.float32)

    def zero_row(r, carry):
        for j in range(D // 16):
            zero_v[r, pl.ds(16 * j, 16)] = z
        return carry

    lax.fori_loop(0, ZR, zero_row, 0)
    for k in range(pl.cdiv(N_UNITS, NS)):
        u = sid + k * NS

        @pl.when(u < N_UNITS)
        def _():
            r = pl.multiple_of(u * ZR, ZR)
            pltpu.sync_copy(zero_v, acc.at[pl.ds(r, ZR)])

    pltpu.sync_copy(idx_hbm.at[wid], idx_v)
    plsc.subcore_barrier()

    def load(it, buf, sem):
        pltpu.async_copy(src_hbm.at[pl.ds(base + it * C, C)], buf, sem)

    def drain(buf, sem):
        pltpu.make_async_copy(src_hbm.at[pl.ds(0, C)], buf, sem).wait()

    def scatter(it, buf):
        pltpu.sync_copy(buf, acc.at[idx_v.at[it]], add=True)

    def step(it, carry):
        def one(cur_buf, cur_sem, prev_buf, prev_sem):
            drain(cur_buf, cur_sem)

            @pl.when(it + 2 < N_CHUNKS)
            def _():
                load(it + 2, prev_buf, prev_sem)

            scatter(it, cur_buf)

        @pl.when(it % 3 == 0)
        def _():
            one(row_v0, sem0, row_v2, sem2)

        @pl.when(it % 3 == 1)
        def _():
            one(row_v1, sem1, row_v0, sem0)

        @pl.when(it % 3 == 2)
        def _():
            one(row_v2, sem2, row_v1, sem1)

        return carry

    lax.fori_loop(0, N_CHUNKS, step, 0)
    plsc.subcore_barrier()

    for k in range(pl.cdiv(N_UNITS, NS)):
        u = sid + k * NS

        @pl.when(u < N_UNITS)
        def _():
            r = pl.multiple_of(u * ZR, ZR)
            pltpu.sync_copy(acc.at[pl.ds(r, ZR)],
                            out_hbm.at[cid].at[pl.ds(r, ZR)])


@jax.jit
def _sc_scatter(src, idx):
    mesh = plsc.VectorSubcoreMesh(core_axis_name="c", subcore_axis_name="s")
    return pl.kernel(
        _sc_body,
        out_type=jax.ShapeDtypeStruct((NC, N_OUT, D), jnp.float32),
        mesh=mesh,
        scratch_types=[
            pltpu.VMEM((N_CHUNKS, C), jnp.int32),
            pltpu.VMEM((C, D), jnp.float32),
            pltpu.VMEM((C, D), jnp.float32),
            pltpu.VMEM((C, D), jnp.float32),
            pltpu.VMEM_SHARED((N_OUT, D), jnp.float32),
            pltpu.SemaphoreType.DMA,
            pltpu.SemaphoreType.DMA,
            pltpu.SemaphoreType.DMA,
        ],
    )(src, idx.reshape(NC * NS, N_CHUNKS, C))


def _combine_body(p_ref, o_ref):
    o_ref[...] = p_ref[0] + p_ref[1]


@jax.jit
def _combine(partials):
    return pl.pallas_call(
        _combine_body,
        out_shape=jax.ShapeDtypeStruct((N_OUT, D), jnp.float32),
    )(partials)


def kernel(src, index, dim_size):
    idx = index.astype(jnp.int32)
    partials = _sc_scatter(src, idx)
    return _combine(partials)


# speedup vs baseline: 8.4600x; 1.0016x over previous
import functools

import jax
import jax.numpy as jnp
from jax import lax
from jax.experimental import pallas as pl
from jax.experimental.pallas import tpu as pltpu
from jax.experimental.pallas import tpu_sc as plsc

N_EDGES = 320000
D = 128
N_OUT = 10000
NC = 2
NS = 16
EDGES_PER_TILE = N_EDGES // (NC * NS)
C = 80
N_CHUNKS = EDGES_PER_TILE // C
ZR = 80
N_UNITS = N_OUT // ZR


def _sc_body(src_hbm, idx_hbm, out_hbm, idx_v, row_v0, row_v1, row_v2,
             acc, sem0, sem1, sem2, wsem):
    cid = lax.axis_index("c")
    sid = lax.axis_index("s")
    wid = cid * NS + sid
    base = wid * EDGES_PER_TILE

    pltpu.async_copy(src_hbm.at[pl.ds(base, C)], row_v0, sem0)
    pltpu.async_copy(src_hbm.at[pl.ds(base + C, C)], row_v1, sem1)

    zero_v = row_v2
    z = jnp.zeros((16,), jnp.float32)

    def zero_row(r, carry):
        for j in range(D // 16):
            zero_v[r, pl.ds(16 * j, 16)] = z
        return carry

    lax.fori_loop(0, ZR, zero_row, 0)
    for k in range(pl.cdiv(N_UNITS, NS)):
        u = sid + k * NS

        @pl.when(u < N_UNITS)
        def _():
            r = pl.multiple_of(u * ZR, ZR)
            pltpu.async_copy(zero_v, acc.at[pl.ds(r, ZR)], wsem)

    for k in range(pl.cdiv(N_UNITS, NS)):
        u = sid + k * NS

        @pl.when(u < N_UNITS)
        def _():
            r = pl.multiple_of(u * ZR, ZR)
            pltpu.make_async_copy(zero_v, acc.at[pl.ds(r, ZR)], wsem).wait()

    pltpu.sync_copy(idx_hbm.at[wid], idx_v)
    plsc.subcore_barrier()

    def load(it, buf, sem):
        pltpu.async_copy(src_hbm.at[pl.ds(base + it * C, C)], buf, sem)

    def drain(buf, sem):
        pltpu.make_async_copy(src_hbm.at[pl.ds(0, C)], buf, sem).wait()

    def scatter(it, buf):
        pltpu.sync_copy(buf, acc.at[idx_v.at[it]], add=True)

    def step(it, carry):
        def one(cur_buf, cur_sem, prev_buf, prev_sem):
            drain(cur_buf, cur_sem)

            @pl.when(it + 2 < N_CHUNKS)
            def _():
                load(it + 2, prev_buf, prev_sem)

            scatter(it, cur_buf)

        @pl.when(it % 3 == 0)
        def _():
            one(row_v0, sem0, row_v2, sem2)

        @pl.when(it % 3 == 1)
        def _():
            one(row_v1, sem1, row_v0, sem0)

        @pl.when(it % 3 == 2)
        def _():
            one(row_v2, sem2, row_v1, sem1)

        return carry

    lax.fori_loop(0, N_CHUNKS, step, 0)
    plsc.subcore_barrier()

    for k in range(pl.cdiv(N_UNITS, NS)):
        u = sid + k * NS

        @pl.when(u < N_UNITS)
        def _():
            r = pl.multiple_of(u * ZR, ZR)
            pltpu.async_copy(acc.at[pl.ds(r, ZR)],
                             out_hbm.at[cid].at[pl.ds(r, ZR)], wsem)

    for k in range(pl.cdiv(N_UNITS, NS)):
        u = sid + k * NS

        @pl.when(u < N_UNITS)
        def _():
            r = pl.multiple_of(u * ZR, ZR)
            pltpu.make_async_copy(acc.at[pl.ds(r, ZR)],
                                  out_hbm.at[cid].at[pl.ds(r, ZR)], wsem).wait()


@jax.jit
def _sc_scatter(src, idx):
    mesh = plsc.VectorSubcoreMesh(core_axis_name="c", subcore_axis_name="s")
    return pl.kernel(
        _sc_body,
        out_type=jax.ShapeDtypeStruct((NC, N_OUT, D), jnp.float32),
        mesh=mesh,
        scratch_types=[
            pltpu.VMEM((N_CHUNKS, C), jnp.int32),
            pltpu.VMEM((C, D), jnp.float32),
            pltpu.VMEM((C, D), jnp.float32),
            pltpu.VMEM((C, D), jnp.float32),
            pltpu.VMEM_SHARED((N_OUT, D), jnp.float32),
            pltpu.SemaphoreType.DMA,
            pltpu.SemaphoreType.DMA,
            pltpu.SemaphoreType.DMA,
            pltpu.SemaphoreType.DMA,
        ],
    )(src, idx.reshape(NC * NS, N_CHUNKS, C))


def _combine_body(p_ref, o_ref):
    o_ref[...] = p_ref[0] + p_ref[1]


@jax.jit
def _combine(partials):
    return pl.pallas_call(
        _combine_body,
        out_shape=jax.ShapeDtypeStruct((N_OUT, D), jnp.float32),
    )(partials)


def kernel(src, index, dim_size):
    idx = index.astype(jnp.int32)
    partials = _sc_scatter(src, idx)
    return _combine(partials)
